# Initial kernel scaffold; baseline (speedup 1.0000x reference)
import jax
import jax.numpy as jnp
from jax.experimental import pallas as pl


def kernel(edge_index, num_nodes, W, b):
    raise NotImplementedError("write your pallas kernel here")


# speedup vs baseline: 110.3751x; 110.3751x over previous
import functools

import jax
import jax.numpy as jnp
from jax import lax
from jax.experimental import pallas as pl
from jax.experimental.pallas import tpu as pltpu
from jax.experimental.pallas import tpu_sc as plsc

N_NODES = 100000
WALK_LENGTH = 16
EMBED_DIM = 16

NTILES = 16
L = 16
NPAD = 102400
SLICE = NPAD // NTILES
CHUNK = 2048
VPC = CHUNK // L
NCH = 98
EPT = CHUNK * NCH
E_PAD = EPT * NTILES
STEPS = WALK_LENGTH - 1
NVS = SLICE // L


def _rsqrt_nr(d):
    i = lax.bitcast_convert_type(d, jnp.int32)
    i = jnp.int32(0x5F3759DF) - (i >> 1)
    y = lax.bitcast_convert_type(i, jnp.float32)
    for _ in range(3):
        y = y * (1.5 - 0.5 * d * y * y)
    return y


def _walk_body(row_hbm, col_hbm, probs_hbm, q_hbm,
               q_local, rowb, colb, valb, accs, dinvs, acc_sh):
    tid = lax.axis_index("s")
    nbase = tid * SLICE
    ebase = tid * EPT
    cbase = tid * NCH
    zeros16 = jnp.zeros((L,), jnp.float32)
    inv_n = jnp.float32(1.0 / N_NODES)

    def _zero_slice(i, _):
        accs[pl.ds(i * L, L)] = zeros16
        return 0

    def _zero_acc_and_publish():
        lax.fori_loop(0, NVS, _zero_slice, 0)
        pltpu.sync_copy(accs, acc_sh.at[pl.ds(nbase, SLICE)])

    _zero_acc_and_publish()

    def _ones_body(r, _):
        for j in range(VPC // L):
            valb[pl.ds((r * (VPC // L) + j) * L, L)] = zeros16 + 1.0
        return 0
    lax.fori_loop(0, L, _ones_body, 0)

    plsc.subcore_barrier()

    def _deg_body(c, _):
        eb = pl.multiple_of(ebase + c * CHUNK, 8)
        pltpu.sync_copy(col_hbm.at[pl.ds(eb, CHUNK)], colb)
        pltpu.sync_copy(valb, acc_sh.at[colb], add=True)
        return 0
    lax.fori_loop(0, NCH, _deg_body, 0)
    plsc.subcore_barrier()

    pltpu.sync_copy(acc_sh.at[pl.ds(nbase, SLICE)], accs)

    def _init_body(i, _):
        deg = accs[pl.ds(i * L, L)] + 1.0
        dinv = _rsqrt_nr(deg)
        dinvs[pl.ds(i * L, L)] = dinv
        accs[pl.ds(i * L, L)] = zeros16 + inv_n
        return 0
    lax.fori_loop(0, NVS, _init_body, 0)
    pltpu.sync_copy(accs, probs_hbm.at[pl.ds(nbase, SLICE)])

    def _q0_body(i, _):
        accs[pl.ds(i * L, L)] = dinvs[pl.ds(i * L, L)] * inv_n
        return 0
    lax.fori_loop(0, NVS, _q0_body, 0)
    pltpu.sync_copy(accs, q_hbm.at[pl.ds(nbase, SLICE)])
    _zero_acc_and_publish()
    plsc.subcore_barrier()
    pltpu.sync_copy(q_hbm, q_local)

    def _step_body(t, _):
        def _chunk_body(c, _c):
            eb = pl.multiple_of(ebase + c * CHUNK, 8)
            pltpu.sync_copy(row_hbm.at[pl.ds(eb, CHUNK)], rowb)
            pltpu.sync_copy(col_hbm.at[pl.ds(eb, CHUNK)], colb)

            def _gat_body(r, _g):
                for j in range(VPC // L):
                    o = r * VPC + j * L
                    idx = rowb[pl.ds(o, L)]
                    valb[pl.ds(o, L)] = plsc.load_gather(q_local, [idx])
                return 0
            lax.fori_loop(0, L, _gat_body, 0)
            pltpu.sync_copy(valb, acc_sh.at[colb], add=True)
            return 0
        lax.fori_loop(0, NCH, _chunk_body, 0)
        plsc.subcore_barrier()

        pltpu.sync_copy(acc_sh.at[pl.ds(nbase, SLICE)], accs)

        def _upd_body(i, _u):
            s = accs[pl.ds(i * L, L)]
            qv = q_local[pl.ds(nbase + i * L, L)]
            dv = dinvs[pl.ds(i * L, L)]
            pv = qv / dv
            pn = 0.9 * (dv * (s + qv)) + 0.1 * pv
            accs[pl.ds(i * L, L)] = pn
            return 0
        lax.fori_loop(0, NVS, _upd_body, 0)

        off = pl.multiple_of((t + 1) * NPAD + nbase, 8)
        pltpu.sync_copy(accs, probs_hbm.at[pl.ds(off, SLICE)])

        def _qn_body(i, _u):
            accs[pl.ds(i * L, L)] = accs[pl.ds(i * L, L)] * dinvs[pl.ds(i * L, L)]
            return 0
        lax.fori_loop(0, NVS, _qn_body, 0)
        pltpu.sync_copy(accs, q_hbm.at[pl.ds(nbase, SLICE)])
        _zero_acc_and_publish()
        plsc.subcore_barrier()
        pltpu.sync_copy(q_hbm, q_local)
        return 0
    lax.fori_loop(0, STEPS, _step_body, 0)


_walk = functools.partial(
    pl.kernel,
    out_type=[
        jax.ShapeDtypeStruct((WALK_LENGTH * NPAD,), jnp.float32),
        jax.ShapeDtypeStruct((NPAD,), jnp.float32),
    ],
    mesh=plsc.VectorSubcoreMesh(
        core_axis_name="c", subcore_axis_name="s", num_cores=1),
    compiler_params=pltpu.CompilerParams(needs_layout_passes=False),
    scratch_types=[
        pltpu.VMEM((NPAD,), jnp.float32),
        pltpu.VMEM((CHUNK,), jnp.int32),
        pltpu.VMEM((CHUNK,), jnp.int32),
        pltpu.VMEM((CHUNK,), jnp.float32),
        pltpu.VMEM((SLICE,), jnp.float32),
        pltpu.VMEM((SLICE,), jnp.float32),
        pltpu.VMEM_SHARED((NPAD,), jnp.float32),
    ],
)(_walk_body)


BN = 2048


def _linear_body(p_ref, w_ref, b_ref, o_ref):
    o_ref[...] = lax.dot_general(
        p_ref[...], w_ref[...], (((0,), (1,)), ((), ())),
        preferred_element_type=jnp.float32) + b_ref[...]


def _linear(probs2d, W, b2d):
    return pl.pallas_call(
        _linear_body,
        grid=(NPAD // BN,),
        in_specs=[
            pl.BlockSpec((WALK_LENGTH, BN), lambda i: (0, i)),
            pl.BlockSpec((EMBED_DIM, WALK_LENGTH), lambda i: (0, 0)),
            pl.BlockSpec((1, EMBED_DIM), lambda i: (0, 0)),
        ],
        out_specs=pl.BlockSpec((BN, EMBED_DIM), lambda i: (i, 0)),
        out_shape=jax.ShapeDtypeStruct((NPAD, EMBED_DIM), jnp.float32),
    )(probs2d, W, b2d)


def kernel(edge_index, num_nodes, W, b):
    ei = edge_index.astype(jnp.int32)
    row, col = ei[0], ei[1]
    pad = E_PAD - row.shape[0]
    row_p = jnp.concatenate([row, jnp.zeros((pad,), jnp.int32)])
    col_p = jnp.concatenate([col, jnp.full((pad,), N_NODES, jnp.int32)])
    probs_flat, _ = _walk(row_p, col_p)
    probs2d = probs_flat.reshape(WALK_LENGTH, NPAD)
    out = _linear(probs2d, W.astype(jnp.float32),
                  b.astype(jnp.float32).reshape(1, EMBED_DIM))
    return out[:N_NODES]
